# baseline (speedup 1.0000x reference)
import jax
import jax.numpy as jnp
from jax.experimental import pallas as pl
from jax.experimental.pallas import tpu as pltpu

_IN = 64
_OUT = 2

_TILE_B = 32768
_MIN_SPLIT = 256


def _linear_t_body(x_ref, w_ref, b_ref, o_ref):
    acc = jax.lax.dot_general(
        w_ref[...], x_ref[...],
        dimension_numbers=(((1,), (1,)), ((), ())),
        preferred_element_type=jnp.float32,
    )
    o_ref[...] = (acc + b_ref[...]).astype(o_ref.dtype)


def kernel(x, weight, bias):
    B = x.shape[0]
    dtype = x.dtype

    if B >= 2 * _TILE_B:
        tile = _TILE_B
    elif B >= _MIN_SPLIT:
        tile = ((pl.cdiv(B, 2) + 7) // 8) * 8
    else:
        tile = B
    grid = (pl.cdiv(B, tile),)

    b2d = bias.astype(dtype).reshape(_OUT, 1)

    out_t = pl.pallas_call(
        _linear_t_body,
        out_shape=jax.ShapeDtypeStruct((_OUT, B), dtype),
        grid=grid,
        in_specs=[
            pl.BlockSpec((tile, _IN), lambda i: (i, 0)),
            pl.BlockSpec((_OUT, _IN), lambda i: (0, 0)),
            pl.BlockSpec((_OUT, 1), lambda i: (0, 0)),
        ],
        out_specs=pl.BlockSpec((_OUT, tile), lambda i: (0, i)),
        compiler_params=pltpu.CompilerParams(
            dimension_semantics=("parallel",),
        ),
    )(x, weight.astype(dtype), b2d)

    return out_t.T


# speedup vs baseline: 1.0220x; 1.0220x over previous
import jax
import jax.numpy as jnp
from jax.experimental import pallas as pl
from jax.experimental.pallas import tpu as pltpu

_IN = 64
_OUT = 2

_TILE_B = 16384
_MIN_SPLIT = 256


def _linear_t_body(x_ref, w_ref, b_ref, o_ref):
    acc = jax.lax.dot_general(
        w_ref[...], x_ref[...],
        dimension_numbers=(((1,), (1,)), ((), ())),
        preferred_element_type=jnp.float32,
    )
    o_ref[...] = (acc + b_ref[...]).astype(o_ref.dtype)


def kernel(x, weight, bias):
    B = x.shape[0]
    dtype = x.dtype

    if B >= 2 * _TILE_B:
        tile = _TILE_B
    elif B >= _MIN_SPLIT:
        tile = ((pl.cdiv(B, 2) + 7) // 8) * 8
    else:
        tile = B
    grid = (pl.cdiv(B, tile),)

    b2d = bias.astype(dtype).reshape(_OUT, 1)

    out_t = pl.pallas_call(
        _linear_t_body,
        out_shape=jax.ShapeDtypeStruct((_OUT, B), dtype),
        grid=grid,
        in_specs=[
            pl.BlockSpec((tile, _IN), lambda i: (i, 0)),
            pl.BlockSpec((_OUT, _IN), lambda i: (0, 0)),
            pl.BlockSpec((_OUT, 1), lambda i: (0, 0)),
        ],
        out_specs=pl.BlockSpec((_OUT, tile), lambda i: (0, i)),
        compiler_params=pltpu.CompilerParams(
            dimension_semantics=("parallel",),
        ),
    )(x, weight.astype(dtype), b2d)

    return out_t.T
